# baseline (speedup 1.0000x reference)
import functools

import jax
import jax.numpy as jnp
from jax import lax
from jax.experimental import pallas as pl
from jax.experimental.pallas import tpu as pltpu
from jax.experimental.pallas import tpu_sc as plsc

_IDX = 13
_B, _F, _D = 16384, 26, 64
_NW = 32
_CPW = _B // _NW
_NBUF = 2
_CHUNK = _CPW // _NBUF


def _body(in_hbm, out_hbm, buf0, buf1, in_sem0, in_sem1, out_sem0, out_sem1):
    wid = lax.axis_index("s") * 2 + lax.axis_index("c")
    base = wid * _CPW

    def in_copy(i, buf, sem):
        return pltpu.make_async_copy(
            in_hbm.at[_IDX, :, pl.ds(base + i * _CHUNK, _CHUNK)], buf, sem
        )

    def out_copy(i, buf, sem):
        return pltpu.make_async_copy(
            buf, out_hbm.at[:, pl.ds(base + i * _CHUNK, _CHUNK)], sem
        )

    in_copy(0, buf0, in_sem0).start()
    in_copy(1, buf1, in_sem1).start()
    in_copy(0, buf0, in_sem0).wait()
    out_copy(0, buf0, out_sem0).start()
    in_copy(1, buf1, in_sem1).wait()
    out_copy(1, buf1, out_sem1).start()
    out_copy(0, buf0, out_sem0).wait()
    out_copy(1, buf1, out_sem1).wait()


def kernel(inputs):
    plane_major = jnp.transpose(inputs, (1, 2, 0))
    mesh = plsc.VectorSubcoreMesh(core_axis_name="c", subcore_axis_name="s")
    run = functools.partial(
        pl.kernel,
        mesh=mesh,
        out_type=jax.ShapeDtypeStruct((_D, _B), jnp.float32),
        scratch_types=[
            pltpu.VMEM((_D, _CHUNK), jnp.float32),
            pltpu.VMEM((_D, _CHUNK), jnp.float32),
            pltpu.SemaphoreType.DMA,
            pltpu.SemaphoreType.DMA,
            pltpu.SemaphoreType.DMA,
            pltpu.SemaphoreType.DMA,
        ],
        compiler_params=pltpu.CompilerParams(
            skip_device_barrier=True,
            disable_bounds_checks=True,
            disable_semaphore_checks=True,
        ),
    )(_body)
    return run(plane_major).T


# speedup vs baseline: 1.0075x; 1.0075x over previous
import functools

import jax
import jax.numpy as jnp
from jax import lax
from jax.experimental import pallas as pl
from jax.experimental.pallas import tpu as pltpu
from jax.experimental.pallas import tpu_sc as plsc

_IDX = 13
_B, _F, _D = 16384, 26, 64
_NW = 32
_CPW = _B // _NW


def _body(in_hbm, out_hbm, buf_v):
    wid = lax.axis_index("s") * 2 + lax.axis_index("c")
    base = wid * _CPW
    pltpu.sync_copy(in_hbm.at[_IDX, :, pl.ds(base, _CPW)], buf_v)
    pltpu.sync_copy(buf_v, out_hbm.at[:, pl.ds(base, _CPW)])


def kernel(inputs):
    plane_major = jnp.transpose(inputs, (1, 2, 0))
    mesh = plsc.VectorSubcoreMesh(core_axis_name="c", subcore_axis_name="s")
    run = functools.partial(
        pl.kernel,
        mesh=mesh,
        out_type=jax.ShapeDtypeStruct((_D, _B), jnp.float32),
        scratch_types=[
            pltpu.VMEM((_D, _CPW), jnp.float32),
        ],
        compiler_params=pltpu.CompilerParams(
            skip_device_barrier=True,
            disable_bounds_checks=True,
            disable_semaphore_checks=True,
        ),
    )(_body)
    return run(plane_major).T
